# baseline (speedup 1.0000x reference)
import functools

import jax
import jax.numpy as jnp
from jax import lax
from jax.experimental import pallas as pl
from jax.experimental.pallas import tpu as pltpu
from jax.experimental.pallas import tpu_sc as plsc

B, T, D = 4096, 200, 64

_NC, _NS = 2, 16
_NW = _NC * _NS
_BPW = B // _NW
_L = 16


@functools.cache
def _make_sc_gather():
    mesh = plsc.VectorSubcoreMesh(core_axis_name="c", subcore_axis_name="s")

    @functools.partial(
        pl.kernel,
        mesh=mesh,
        out_type=jax.ShapeDtypeStruct((B, D), jnp.float32),
        scratch_types=[
            pltpu.VMEM((_BPW,), jnp.int32),
            pltpu.VMEM((_BPW, D), jnp.float32),
            pltpu.SemaphoreType.DMA,
        ],
    )
    def sc_gather(table_hbm, idx_hbm, out_hbm, idx_v, rows_v, sem):
        wid = lax.axis_index("s") * _NC + lax.axis_index("c")
        base = wid * _BPW
        pltpu.sync_copy(idx_hbm.at[pl.ds(base, _BPW)], idx_v)
        for j in range(_BPW // _L):
            v = idx_v[pl.ds(j * _L, _L)]
            copies = []
            for k in range(_L):
                copies.append(
                    pltpu.async_copy(
                        table_hbm.at[pl.ds(v[k], 1)],
                        rows_v.at[pl.ds(j * _L + k, 1)],
                        sem,
                    )
                )
            for c in copies:
                c.wait()
        pltpu.sync_copy(rows_v, out_hbm.at[pl.ds(base, _BPW)])

    return sc_gather


_BB = 128


def _tc_body(x_ref, e_ref, o_ref):
    e = jnp.broadcast_to(e_ref[...][:, None, :], (_BB, T, D))
    o_ref[:, :, 0:D] = x_ref[...]
    o_ref[:, :, D:2 * D] = e


def kernel(x, ticker, embed):
    idx = ticker.astype(jnp.int32)
    e = _make_sc_gather()(embed, idx)
    return pl.pallas_call(
        _tc_body,
        grid=(B // _BB,),
        in_specs=[
            pl.BlockSpec((_BB, T, D), lambda i: (i, 0, 0)),
            pl.BlockSpec((_BB, D), lambda i: (i, 0)),
        ],
        out_specs=pl.BlockSpec((_BB, T, 2 * D), lambda i: (i, 0, 0)),
        out_shape=jax.ShapeDtypeStruct((B, T, 2 * D), jnp.float32),
    )(x, e)


# speedup vs baseline: 1.5964x; 1.5964x over previous
import functools

import jax
import jax.numpy as jnp
from jax import lax
from jax.experimental import pallas as pl
from jax.experimental.pallas import tpu as pltpu
from jax.experimental.pallas import tpu_sc as plsc

B, T, D = 4096, 200, 64

_NC, _NS = 2, 16
_NW = _NC * _NS
_BPW = B // _NW
_L = 16


@functools.cache
def _make_sc_gather():
    mesh = plsc.VectorSubcoreMesh(core_axis_name="c", subcore_axis_name="s")

    @functools.partial(
        pl.kernel,
        mesh=mesh,
        out_type=jax.ShapeDtypeStruct((B, D), jnp.float32),
        scratch_types=[
            pltpu.VMEM((_BPW,), jnp.int32),
            pltpu.VMEM((_BPW, D), jnp.float32),
            pltpu.SemaphoreType.DMA,
        ],
    )
    def sc_gather(table_hbm, idx_hbm, out_hbm, idx_v, rows_v, sem):
        wid = lax.axis_index("s") * _NC + lax.axis_index("c")
        base = wid * _BPW
        pltpu.sync_copy(idx_hbm.at[pl.ds(base, _BPW)], idx_v)
        for j in range(_BPW // _L):
            v = idx_v[pl.ds(j * _L, _L)]
            copies = []
            for k in range(_L):
                copies.append(
                    pltpu.async_copy(
                        table_hbm.at[pl.ds(v[k], 1)],
                        rows_v.at[pl.ds(j * _L + k, 1)],
                        sem,
                    )
                )
            for c in copies:
                c.wait()
        pltpu.sync_copy(rows_v, out_hbm.at[pl.ds(base, _BPW)])

    return sc_gather


_BB = 128


def _tc_body(xt_ref, e_ref, o_ref):
    xt = jnp.transpose(xt_ref[...], (2, 0, 1))
    o_ref[:, :, 0:D] = xt
    o_ref[:, :, D:2 * D] = jnp.broadcast_to(
        e_ref[...][:, None, :], (_BB, T, D))


def kernel(x, ticker, embed):
    idx = ticker.astype(jnp.int32)
    x_t = jnp.transpose(x, (1, 2, 0))
    e = _make_sc_gather()(embed, idx)
    return pl.pallas_call(
        _tc_body,
        grid=(B // _BB,),
        in_specs=[
            pl.BlockSpec((T, D, _BB), lambda i: (0, 0, i)),
            pl.BlockSpec((_BB, D), lambda i: (i, 0)),
        ],
        out_specs=pl.BlockSpec((_BB, T, 2 * D), lambda i: (i, 0, 0)),
        out_shape=jax.ShapeDtypeStruct((B, T, 2 * D), jnp.float32),
    )(x_t, e)
